# baseline (speedup 1.0000x reference)
import functools

import jax
import jax.numpy as jnp
from jax import lax
from jax.experimental import pallas as pl
from jax.experimental.pallas import tpu as pltpu
from jax.experimental.pallas import tpu_sc as plsc

B = 16384
USER_DIM = 32
ITEM_DIM = 64
HIDDEN = 64
MARGIN = 1.0

NC = 2
NS = 16
NW = NC * NS
BPW = B // NW
CHUNK = 128
NCHUNK = BPW // CHUNK


def _sc_gather(uidx2d, pidx2d, nidx2d, user_table, item_table):
    mesh = plsc.VectorSubcoreMesh(core_axis_name="c", subcore_axis_name="s")

    @functools.partial(
        pl.kernel,
        out_type=[
            jax.ShapeDtypeStruct((B, USER_DIM), jnp.float32),
            jax.ShapeDtypeStruct((B, ITEM_DIM), jnp.float32),
            jax.ShapeDtypeStruct((B, ITEM_DIM), jnp.float32),
        ],
        mesh=mesh,
        compiler_params=pltpu.CompilerParams(use_tc_tiling_on_sc=False),
        scratch_types=[
            pltpu.VMEM((NCHUNK, CHUNK), jnp.int32),
            pltpu.VMEM((NCHUNK, CHUNK), jnp.int32),
            pltpu.VMEM((NCHUNK, CHUNK), jnp.int32),
            pltpu.VMEM((BPW, USER_DIM), jnp.float32),
            pltpu.VMEM((BPW, ITEM_DIM), jnp.float32),
            pltpu.VMEM((BPW, ITEM_DIM), jnp.float32),
            pltpu.SemaphoreType.DMA,
            pltpu.SemaphoreType.DMA,
            pltpu.SemaphoreType.DMA,
        ],
    )
    def k(uidx_hbm, pidx_hbm, nidx_hbm, utab_hbm, itab_hbm,
          ue_hbm, pe_hbm, ne_hbm,
          uidx_v, pidx_v, nidx_v, urows, prows, nrows, su, sp, sn):
        wid = lax.axis_index("s") * NC + lax.axis_index("c")
        row0 = wid * NCHUNK
        base = wid * BPW

        pltpu.sync_copy(uidx_hbm.at[pl.ds(row0, NCHUNK)], uidx_v)
        pltpu.sync_copy(pidx_hbm.at[pl.ds(row0, NCHUNK)], pidx_v)
        pltpu.sync_copy(nidx_hbm.at[pl.ds(row0, NCHUNK)], nidx_v)

        waits = []
        for j in range(NCHUNK):
            waits.append(pltpu.async_copy(
                utab_hbm.at[uidx_v.at[j]], urows.at[pl.ds(j * CHUNK, CHUNK)], su))
            waits.append(pltpu.async_copy(
                itab_hbm.at[pidx_v.at[j]], prows.at[pl.ds(j * CHUNK, CHUNK)], sp))
            waits.append(pltpu.async_copy(
                itab_hbm.at[nidx_v.at[j]], nrows.at[pl.ds(j * CHUNK, CHUNK)], sn))
        for w in waits:
            w.wait()

        pltpu.sync_copy(urows, ue_hbm.at[pl.ds(base, BPW)])
        pltpu.sync_copy(prows, pe_hbm.at[pl.ds(base, BPW)])
        pltpu.sync_copy(nrows, ne_hbm.at[pl.ds(base, BPW)])

    return k(uidx2d, pidx2d, nidx2d, user_table, item_table)


def _mlp_loss_kernel(ue_ref, pe_ref, ne_ref, w1u_ref, w1i_ref, b1_ref,
                     w2t_ref, b2_ref, out_ref):
    ue = ue_ref[...]
    u = jnp.dot(ue, w1u_ref[...], preferred_element_type=jnp.float32)
    b1 = b1_ref[...]
    hp = jnp.maximum(
        u + jnp.dot(pe_ref[...], w1i_ref[...],
                    preferred_element_type=jnp.float32) + b1, 0.0)
    hn = jnp.maximum(
        u + jnp.dot(ne_ref[...], w1i_ref[...],
                    preferred_element_type=jnp.float32) + b1, 0.0)
    w2t = w2t_ref[...]
    b2 = b2_ref[0, 0]
    op = jnp.maximum(jnp.sum(hp * w2t, axis=1, keepdims=True) + b2, 0.0)
    on = jnp.maximum(jnp.sum(hn * w2t, axis=1, keepdims=True) + b2, 0.0)
    out_ref[0, 0] = jnp.sum(jnp.maximum(on - op + MARGIN, 0.0)) / B


def kernel(user_input, pos_item_input, neg_item_input, user_table,
           item_table, W1, b1, W2, b2):
    uidx2d = user_input.reshape(B // CHUNK, CHUNK)
    pidx2d = pos_item_input.reshape(B // CHUNK, CHUNK)
    nidx2d = neg_item_input.reshape(B // CHUNK, CHUNK)

    ue, pe, ne = _sc_gather(uidx2d, pidx2d, nidx2d, user_table, item_table)

    w1u = W1[:USER_DIM]
    w1i = W1[USER_DIM:]
    b1r = b1.reshape(1, HIDDEN)
    w2t = W2.reshape(1, HIDDEN)
    b2r = b2.reshape(1, 1)

    loss = pl.pallas_call(
        _mlp_loss_kernel,
        out_shape=jax.ShapeDtypeStruct((1, 1), jnp.float32),
        out_specs=pl.BlockSpec(memory_space=pltpu.SMEM),
    )(ue, pe, ne, w1u, w1i, b1r, w2t, b2r)
    return loss[0, 0]
